# baseline (speedup 1.0000x reference)
import functools

import jax
import jax.numpy as jnp
from jax import lax
from jax.experimental import pallas as pl
from jax.experimental.pallas import tpu as pltpu
from jax.experimental.pallas import tpu_sc as plsc

BATCH = 16384
D = 64
L = 16
NC = 2
NS = 16
NW = NC * NS
BPW = BATCH // NW
CHUNK = 128
NCHUNK = BPW // CHUNK


def _nr_rsqrt(x):
    i = lax.bitcast_convert_type(x, jnp.int32)
    i = jnp.int32(0x5F3759DF) - lax.shift_right_logical(i, 1)
    y = lax.bitcast_convert_type(i, jnp.float32)
    for _ in range(3):
        y = y * (jnp.float32(1.5) - jnp.float32(0.5) * x * y * y)
    return y


def _sc_body(uids_hbm, rids_hbm, utab_hbm, rtab_hbm, out_hbm,
             uidx_v, ridx_v, urows_v, rrows_v, out_v, sem):
    wid = lax.axis_index("s") * NC + lax.axis_index("c")
    base = wid * BPW

    pltpu.sync_copy(uids_hbm.at[wid], uidx_v)
    pltpu.sync_copy(rids_hbm.at[wid], ridx_v)

    copies = []
    for j in range(NCHUNK):
        copies.append(pltpu.async_copy(
            utab_hbm.at[uidx_v.at[j]], urows_v.at[pl.ds(j * CHUNK, CHUNK)], sem))
        copies.append(pltpu.async_copy(
            rtab_hbm.at[ridx_v.at[j]], rrows_v.at[pl.ds(j * CHUNK, CHUNK)], sem))
    for c in copies:
        c.wait()

    lanes = lax.iota(jnp.int32, L)
    eps2 = jnp.float32(1e-16)

    def group(g, carry):
        row = g * L + lanes
        dot = jnp.zeros((L,), jnp.float32)
        u2 = jnp.zeros((L,), jnp.float32)
        r2 = jnp.zeros((L,), jnp.float32)
        for d in range(D):
            col = jnp.full((L,), d, jnp.int32)
            uc = plsc.load_gather(urows_v, [row, col])
            rc = plsc.load_gather(rrows_v, [row, col])
            dot = dot + uc * rc
            u2 = u2 + uc * uc
            r2 = r2 + rc * rc
        sim = dot * _nr_rsqrt(jnp.maximum(u2, eps2)) * _nr_rsqrt(jnp.maximum(r2, eps2))
        out_v[pl.ds(g * L, L)] = sim
        return carry

    lax.fori_loop(0, BPW // L, group, 0)
    pltpu.sync_copy(out_v, out_hbm.at[pl.ds(base, BPW)])


def kernel(user_ids, reel_ids, user_table, reel_table):
    uids = user_ids.astype(jnp.int32).reshape(NW, NCHUNK, CHUNK)
    rids = reel_ids.astype(jnp.int32).reshape(NW, NCHUNK, CHUNK)
    mesh = plsc.VectorSubcoreMesh(core_axis_name="c", subcore_axis_name="s")
    fn = functools.partial(
        pl.kernel,
        mesh=mesh,
        compiler_params=pltpu.CompilerParams(needs_layout_passes=False, use_tc_tiling_on_sc=False),
        out_type=jax.ShapeDtypeStruct((BATCH,), jnp.float32),
        scratch_types=[
            pltpu.VMEM((NCHUNK, CHUNK), jnp.int32),
            pltpu.VMEM((NCHUNK, CHUNK), jnp.int32),
            pltpu.VMEM((BPW, D), jnp.float32),
            pltpu.VMEM((BPW, D), jnp.float32),
            pltpu.VMEM((BPW,), jnp.float32),
            pltpu.SemaphoreType.DMA,
        ],
    )(_sc_body)
    return fn(uids, rids, user_table, reel_table)


# speedup vs baseline: 1.5574x; 1.5574x over previous
import functools

import jax
import jax.numpy as jnp
from jax import lax
from jax.experimental import pallas as pl
from jax.experimental.pallas import tpu as pltpu
from jax.experimental.pallas import tpu_sc as plsc

BATCH = 16384
D = 64
L = 16
NC = 2
NS = 16
NW = NC * NS
BPW = BATCH // NW
PASS_ROWS = 256
NPASS = BPW // PASS_ROWS
NGP = PASS_ROWS // L


def _nr_rsqrt(x):
    i = lax.bitcast_convert_type(x, jnp.int32)
    i = jnp.int32(0x5F3759DF) - lax.shift_right_logical(i, 1)
    y = lax.bitcast_convert_type(i, jnp.float32)
    for _ in range(3):
        y = y * (jnp.float32(1.5) - jnp.float32(0.5) * x * y * y)
    return y


def _sc_body(uids_hbm, rids_hbm, utab_hbm, rtab_hbm, out_hbm,
             uidx_v, ridx_v, urows_v, rrows_v, out_v, sem):
    wid = lax.axis_index("s") * NC + lax.axis_index("c")
    base = wid * BPW

    pltpu.sync_copy(uids_hbm.at[pl.ds(base, BPW)], uidx_v)
    pltpu.sync_copy(rids_hbm.at[pl.ds(base, BPW)], ridx_v)

    lanes = lax.iota(jnp.int32, L)
    eps2 = jnp.float32(1e-16)

    for p in range(NPASS):
        def fire(t, carry):
            uvec = uidx_v[pl.ds(p * PASS_ROWS + t * L, L)]
            rvec = ridx_v[pl.ds(p * PASS_ROWS + t * L, L)]
            for i in range(L):
                slot = t * L + i
                pltpu.async_copy(utab_hbm.at[uvec[i]], urows_v.at[slot], sem)
                pltpu.async_copy(rtab_hbm.at[rvec[i]], rrows_v.at[slot], sem)
            return carry

        lax.fori_loop(0, NGP, fire, 0)

        def drain(t, carry):
            for _ in range(2 * L):
                pltpu.make_async_copy(utab_hbm.at[0], urows_v.at[0], sem).wait()
            return carry

        lax.fori_loop(0, NGP, drain, 0)

        def group(g, carry):
            row = g * L + lanes
            dot = jnp.zeros((L,), jnp.float32)
            u2 = jnp.zeros((L,), jnp.float32)
            r2 = jnp.zeros((L,), jnp.float32)
            for d in range(D):
                col = jnp.full((L,), d, jnp.int32)
                uc = plsc.load_gather(urows_v, [row, col])
                rc = plsc.load_gather(rrows_v, [row, col])
                dot = dot + uc * rc
                u2 = u2 + uc * uc
                r2 = r2 + rc * rc
            sim = (dot * _nr_rsqrt(jnp.maximum(u2, eps2))
                   * _nr_rsqrt(jnp.maximum(r2, eps2)))
            out_v[pl.ds(p * PASS_ROWS + g * L, L)] = sim
            return carry

        lax.fori_loop(0, NGP, group, 0)

    pltpu.sync_copy(out_v, out_hbm.at[pl.ds(base, BPW)])


def kernel(user_ids, reel_ids, user_table, reel_table):
    uids = user_ids.astype(jnp.int32)
    rids = reel_ids.astype(jnp.int32)
    mesh = plsc.VectorSubcoreMesh(core_axis_name="c", subcore_axis_name="s")
    fn = functools.partial(
        pl.kernel,
        mesh=mesh,
        compiler_params=pltpu.CompilerParams(
            needs_layout_passes=False, use_tc_tiling_on_sc=True),
        out_type=jax.ShapeDtypeStruct((BATCH,), jnp.float32),
        scratch_types=[
            pltpu.VMEM((BPW,), jnp.int32),
            pltpu.VMEM((BPW,), jnp.int32),
            pltpu.VMEM((PASS_ROWS, D), jnp.float32),
            pltpu.VMEM((PASS_ROWS, D), jnp.float32),
            pltpu.VMEM((BPW,), jnp.float32),
            pltpu.SemaphoreType.DMA,
        ],
    )(_sc_body)
    return fn(uids, rids, user_table, reel_table)
